# speedup vs baseline: 4.0331x; 4.0331x over previous
import functools

import jax
import jax.numpy as jnp
from jax import lax
from jax.experimental import pallas as pl
from jax.experimental.pallas import tpu as pltpu
from jax.experimental.pallas import tpu_sc as plsc

LANES = 128
NW = 32


def _sc_aggregate(node, src2d, dst2d, n_acc, gpw):
    n, d = node.shape
    rpt = n_acc // 16
    nzc = rpt // LANES
    mesh = plsc.VectorSubcoreMesh(core_axis_name="c", subcore_axis_name="s")

    @functools.partial(
        pl.kernel,
        out_type=jax.ShapeDtypeStruct((2, n_acc, d), jnp.float32),
        mesh=mesh,
        scratch_types=[
            pltpu.VMEM((LANES,), jnp.int32),
            pltpu.VMEM((LANES,), jnp.int32),
            pltpu.VMEM((LANES, d), jnp.float32),
            pltpu.VMEM_SHARED((n_acc, d), jnp.float32),
            pltpu.SemaphoreType.DMA,
        ],
    )
    def k(node_hbm, src_hbm, dst_hbm, out_hbm, idx_s, idx_d, rows, aggr, sem):
        c = lax.axis_index("c")
        s = lax.axis_index("s")
        wid = c * 16 + s

        def zrow(r, carry):
            for j in range(d // 16):
                rows[r, pl.ds(j * 16, 16)] = jnp.zeros((16,), jnp.float32)
            return carry
        lax.fori_loop(0, LANES, zrow, 0)
        for kk in range(nzc):
            pltpu.sync_copy(rows, aggr.at[pl.ds(s * rpt + kk * LANES, LANES)])
        plsc.subcore_barrier()

        def body(g, carry):
            gb = wid * gpw + g
            pltpu.sync_copy(src_hbm.at[gb], idx_s)
            pltpu.sync_copy(dst_hbm.at[gb], idx_d)
            pltpu.async_copy(node_hbm.at[idx_s], rows, sem).wait()
            pltpu.sync_copy(rows, aggr.at[idx_d], add=True)
            return carry
        lax.fori_loop(0, gpw, body, 0)

        plsc.subcore_barrier()
        pltpu.sync_copy(aggr.at[pl.ds(s * rpt, rpt)],
                        out_hbm.at[c, pl.ds(s * rpt, rpt)])

    return k(node, src2d, dst2d)


def _mlp_block(node_ref, p0_ref, p1_ref, w1_ref, w2_ref, w3_ref, v_ref,
               eps_ref, out_ref):
    def ln(x, g, b):
        mu = jnp.mean(x, axis=-1, keepdims=True)
        var = jnp.mean((x - mu) ** 2, axis=-1, keepdims=True)
        return (x - mu) * lax.rsqrt(var + 1e-5) * g + b

    b1 = v_ref[0:1, :]
    g1 = v_ref[1:2, :]
    be1 = v_ref[2:3, :]
    b2 = v_ref[3:4, :]
    g2 = v_ref[4:5, :]
    be2 = v_ref[5:6, :]
    b3 = v_ref[6:7, :]
    gn = v_ref[7:8, :]
    bn = v_ref[8:9, :]

    h = (1.0 + eps_ref[0]) * node_ref[...] + p0_ref[0] + p1_ref[0]
    x = jnp.dot(h, w1_ref[...], preferred_element_type=jnp.float32) + b1
    x = jnp.maximum(ln(x, g1, be1), 0.0)
    x = jnp.dot(x, w2_ref[...], preferred_element_type=jnp.float32) + b2
    x = jnp.maximum(ln(x, g2, be2), 0.0)
    x = jnp.dot(x, w3_ref[...], preferred_element_type=jnp.float32) + b3
    out_ref[...] = jnp.maximum(ln(x, gn, bn), 0.0)


def kernel(node, edge_index, edge_attr, batch_ptr, W1, b1, g1, be1,
           W2, b2, g2, be2, W3, b3, eps, gN, bN):
    n, d = node.shape
    e = edge_index.shape[1]

    gpw = -(-e // (NW * LANES))
    gt = NW * gpw
    pad = gt * LANES - e
    src = edge_index[0].astype(jnp.int32)
    dst = edge_index[1].astype(jnp.int32)
    if pad:
        src = jnp.concatenate([src, jnp.zeros((pad,), jnp.int32)])
        dst = jnp.concatenate([dst, jnp.full((pad,), n, jnp.int32)])
    src2d = src.reshape(gt, LANES)
    dst2d = dst.reshape(gt, LANES)

    n_acc = -(-(n + 1) // (16 * LANES)) * (16 * LANES)

    partials = _sc_aggregate(node, src2d, dst2d, n_acc, gpw)

    rb = 1000
    grid = (n // rb,)
    vecs = jnp.stack([b1, g1, be1, b2, g2, be2, b3, gN, bN])
    eps2 = jnp.reshape(eps, (1,))

    out = pl.pallas_call(
        _mlp_block,
        grid=grid,
        in_specs=[
            pl.BlockSpec((rb, d), lambda i: (i, 0)),
            pl.BlockSpec((1, rb, d), lambda i: (0, i, 0)),
            pl.BlockSpec((1, rb, d), lambda i: (1, i, 0)),
            pl.BlockSpec((d, d), lambda i: (0, 0)),
            pl.BlockSpec((d, d), lambda i: (0, 0)),
            pl.BlockSpec((d, d), lambda i: (0, 0)),
            pl.BlockSpec((9, d), lambda i: (0, 0)),
            pl.BlockSpec(memory_space=pltpu.SMEM),
        ],
        out_specs=pl.BlockSpec((rb, d), lambda i: (i, 0)),
        out_shape=jax.ShapeDtypeStruct((n, d), jnp.float32),
    )(node, partials, partials, W1, W2, W3, vecs, eps2)
    return out
